# initial kernel scaffold (unmeasured)
import functools

import jax
import jax.numpy as jnp
from jax import lax
from jax.experimental import pallas as pl
from jax.experimental.pallas import tpu as pltpu

N_DEV = 4
SQ = 256
D_MODEL = 1024
HG = 8
DH = 128
SKV = 4096
SCALE = 0.08838834764831843
NEG = jnp.float32(-1e9)


def kernel(x, Wq, K_ext, V_ext, Wo):
    my = lax.axis_index("i")

    x_bf = x[0].astype(jnp.bfloat16)
    wq_bf = Wq.astype(jnp.bfloat16)
    wo_bf = Wo.astype(jnp.bfloat16)
    k_sl = lax.dynamic_slice(K_ext, (0, 0, my * HG, 0), (1, SKV, HG, DH))[0]
    v_sl = lax.dynamic_slice(V_ext, (0, 0, my * HG, 0), (1, SKV, HG, DH))[0]
    k_h = jnp.transpose(k_sl, (1, 0, 2)).astype(jnp.bfloat16)
    v_h = jnp.transpose(v_sl, (1, 0, 2)).astype(jnp.bfloat16)

    def body(x_ref, wq_ref, k_ref, v_ref, wo_ref, out_ref,
             xbuf, partial, rs_sbuf, rs_rbuf,
             ag_ssem, ag_rsem, rs_ssem, rs_rsem):
        my_i = lax.axis_index("i")
        left = (my_i + N_DEV - 1) % N_DEV
        right = (my_i + 1) % N_DEV

        bsem = pltpu.get_barrier_semaphore()
        for nbr in (left, right):
            pl.semaphore_signal(bsem, inc=1, device_id=(nbr,),
                                device_id_type=pl.DeviceIdType.MESH)
        pl.semaphore_wait(bsem, 2)

        def ag_copy(h):
            src = x_ref if h == 0 else xbuf.at[h - 1]
            return pltpu.make_async_remote_copy(
                src_ref=src, dst_ref=xbuf.at[h],
                send_sem=ag_ssem.at[h], recv_sem=ag_rsem.at[h],
                device_id=(right,), device_id_type=pl.DeviceIdType.MESH)

        def rs_copy(t):
            return pltpu.make_async_remote_copy(
                src_ref=rs_sbuf.at[t], dst_ref=rs_rbuf.at[t],
                send_sem=rs_ssem.at[t], recv_sem=rs_rsem.at[t],
                device_id=(right,), device_id_type=pl.DeviceIdType.MESH)

        def compute_partial(slot):
            q = (my_i + (N_DEV - slot)) % N_DEV
            xq = x_ref[:] if slot == 0 else xbuf[slot - 1]
            qm = jnp.dot(xq, wq_ref[:], preferred_element_type=jnp.float32)
            qm = (qm * SCALE).astype(jnp.bfloat16)
            qi = q * SQ + lax.broadcasted_iota(jnp.int32, (SQ, SKV), 0)
            ki = lax.broadcasted_iota(jnp.int32, (SQ, SKV), 1)
            mask = (jnp.abs(qi - ki) <= 128) | (ki < 32) | (qi < 32)
            parts = []
            for h in range(HG):
                qh = qm[:, h * DH:(h + 1) * DH]
                s = lax.dot_general(qh, k_ref[h], (((1,), (1,)), ((), ())),
                                    preferred_element_type=jnp.float32)
                s = jnp.where(mask, s, NEG)
                m = jnp.max(s, axis=1, keepdims=True)
                w = jnp.exp(s - m)
                d = jnp.sum(w, axis=1, keepdims=True)
                w = (w / d).astype(jnp.bfloat16)
                parts.append(jnp.dot(w, v_ref[h],
                                     preferred_element_type=jnp.float32))
            ctx = jnp.concatenate(parts, axis=1).astype(jnp.bfloat16)
            partial[slot] = jnp.dot(ctx, wo_ref[:],
                                    preferred_element_type=jnp.float32)

        ag0 = ag_copy(0)
        ag0.start()
        compute_partial(0)
        ag0.wait()

        ag1 = ag_copy(1)
        ag1.start()
        compute_partial(1)
        rs_sbuf[0] = partial[1].astype(jnp.bfloat16)
        rs0 = rs_copy(0)
        rs0.start()
        ag1.wait()

        ag2 = ag_copy(2)
        ag2.start()
        compute_partial(2)
        rs0.wait()
        partial[2] += rs_rbuf[0].astype(jnp.float32)
        rs_sbuf[1] = partial[2].astype(jnp.bfloat16)
        rs1 = rs_copy(1)
        rs1.start()
        ag2.wait()

        compute_partial(3)
        rs1.wait()
        partial[3] += rs_rbuf[1].astype(jnp.float32)
        rs_sbuf[2] = partial[3].astype(jnp.bfloat16)
        rs2 = rs_copy(2)
        rs2.start()
        rs2.wait()
        out_ref[:] = partial[0] + rs_rbuf[2].astype(jnp.float32)

        @functools.partial(pl.run_scoped, sem2=pltpu.SemaphoreType.REGULAR)
        def _(sem2):
            for nbr in (left, right):
                pl.semaphore_signal(sem2, inc=1, device_id=(nbr,),
                                    device_id_type=pl.DeviceIdType.MESH)
            pl.semaphore_wait(sem2, 2)

    out = pl.pallas_call(
        body,
        out_shape=jax.ShapeDtypeStruct((SQ, D_MODEL), jnp.float32),
        in_specs=[pl.BlockSpec(memory_space=pltpu.VMEM)] * 5,
        out_specs=pl.BlockSpec(memory_space=pltpu.VMEM),
        scratch_shapes=[
            pltpu.VMEM((3, SQ, D_MODEL), jnp.bfloat16),
            pltpu.VMEM((N_DEV, SQ, D_MODEL), jnp.float32),
            pltpu.VMEM((3, SQ, D_MODEL), jnp.bfloat16),
            pltpu.VMEM((3, SQ, D_MODEL), jnp.bfloat16),
            pltpu.SemaphoreType.DMA((3,)),
            pltpu.SemaphoreType.DMA((3,)),
            pltpu.SemaphoreType.DMA((3,)),
            pltpu.SemaphoreType.DMA((3,)),
        ],
        compiler_params=pltpu.CompilerParams(collective_id=0),
    )(x_bf, wq_bf, k_h, v_h, wo_bf)
    return out[None]


# baseline (device time: 136039 ns/iter reference)
import functools

import jax
import jax.numpy as jnp
from jax import lax
from jax.experimental import pallas as pl
from jax.experimental.pallas import tpu as pltpu

N_DEV = 4
SQ = 256
D_MODEL = 1024
HG = 8
DH = 128
SKV = 4096
SCALE = 0.08838834764831843
NEG = -1e9


def kernel(x, Wq, K_ext, V_ext, Wo):
    my = lax.axis_index("i")

    x_bf = x[0].astype(jnp.bfloat16)
    wq_bf = Wq.astype(jnp.bfloat16)
    wo_bf = Wo.astype(jnp.bfloat16)
    k_sl = lax.dynamic_slice(K_ext, (0, 0, my * HG, 0), (1, SKV, HG, DH))[0]
    v_sl = lax.dynamic_slice(V_ext, (0, 0, my * HG, 0), (1, SKV, HG, DH))[0]
    k_h = jnp.transpose(k_sl, (1, 0, 2)).astype(jnp.bfloat16)
    v_h = jnp.transpose(v_sl, (1, 0, 2)).astype(jnp.bfloat16)

    def body(x_ref, wq_ref, k_ref, v_ref, wo_ref, out_ref,
             xbuf, partial, rs_sbuf, rs_rbuf,
             ag_ssem, ag_rsem, rs_ssem, rs_rsem):
        my_i = lax.axis_index("i")
        left = (my_i + N_DEV - 1) % N_DEV
        right = (my_i + 1) % N_DEV

        bsem = pltpu.get_barrier_semaphore()
        for nbr in (left, right):
            pl.semaphore_signal(bsem, inc=1, device_id=(nbr,),
                                device_id_type=pl.DeviceIdType.MESH)
        pl.semaphore_wait(bsem, 2)

        def ag_copy(h):
            src = x_ref if h == 0 else xbuf.at[h - 1]
            return pltpu.make_async_remote_copy(
                src_ref=src, dst_ref=xbuf.at[h],
                send_sem=ag_ssem.at[h], recv_sem=ag_rsem.at[h],
                device_id=(right,), device_id_type=pl.DeviceIdType.MESH)

        def rs_copy(t):
            return pltpu.make_async_remote_copy(
                src_ref=rs_sbuf.at[t], dst_ref=rs_rbuf.at[t],
                send_sem=rs_ssem.at[t], recv_sem=rs_rsem.at[t],
                device_id=(right,), device_id_type=pl.DeviceIdType.MESH)

        def compute_partial(slot):
            q = (my_i + (N_DEV - slot)) % N_DEV
            xq = x_ref[:] if slot == 0 else xbuf[slot - 1]
            qm = jnp.dot(xq, wq_ref[:], preferred_element_type=jnp.float32)
            qm = (qm * SCALE).astype(jnp.bfloat16)
            qi = q * SQ + lax.broadcasted_iota(jnp.int32, (SQ, SKV), 0)
            ki = lax.broadcasted_iota(jnp.int32, (SQ, SKV), 1)
            mask = (jnp.abs(qi - ki) <= 128) | (ki < 32) | (qi < 32)
            parts = []
            for h in range(HG):
                qh = qm[:, h * DH:(h + 1) * DH]
                s = lax.dot_general(qh, k_ref[h], (((1,), (1,)), ((), ())),
                                    preferred_element_type=jnp.float32)
                s = jnp.where(mask, s, jnp.float32(NEG))
                m = jnp.max(s, axis=1, keepdims=True)
                w = jnp.exp(s - m)
                d = jnp.sum(w, axis=1, keepdims=True)
                w = (w / d).astype(jnp.bfloat16)
                parts.append(jnp.dot(w, v_ref[h],
                                     preferred_element_type=jnp.float32))
            ctx = jnp.concatenate(parts, axis=1).astype(jnp.bfloat16)
            partial[slot] = jnp.dot(ctx, wo_ref[:],
                                    preferred_element_type=jnp.float32)

        ag0 = ag_copy(0)
        ag0.start()
        compute_partial(0)
        ag0.wait()

        ag1 = ag_copy(1)
        ag1.start()
        compute_partial(1)
        rs_sbuf[0] = partial[1].astype(jnp.bfloat16)
        rs0 = rs_copy(0)
        rs0.start()
        ag1.wait()

        ag2 = ag_copy(2)
        ag2.start()
        compute_partial(2)
        rs0.wait()
        partial[2] += rs_rbuf[0].astype(jnp.float32)
        rs_sbuf[1] = partial[2].astype(jnp.bfloat16)
        rs1 = rs_copy(1)
        rs1.start()
        ag2.wait()

        compute_partial(3)
        rs1.wait()
        partial[3] += rs_rbuf[1].astype(jnp.float32)
        rs_sbuf[2] = partial[3].astype(jnp.bfloat16)
        rs2 = rs_copy(2)
        rs2.start()
        rs2.wait()
        out_ref[:] = partial[0] + rs_rbuf[2].astype(jnp.float32)

        @functools.partial(pl.run_scoped, sem2=pltpu.SemaphoreType.REGULAR)
        def _(sem2):
            for nbr in (left, right):
                pl.semaphore_signal(sem2, inc=1, device_id=(nbr,),
                                    device_id_type=pl.DeviceIdType.MESH)
            pl.semaphore_wait(sem2, 2)

    out = pl.pallas_call(
        body,
        out_shape=jax.ShapeDtypeStruct((SQ, D_MODEL), jnp.float32),
        in_specs=[pl.BlockSpec(memory_space=pltpu.VMEM)] * 5,
        out_specs=pl.BlockSpec(memory_space=pltpu.VMEM),
        scratch_shapes=[
            pltpu.VMEM((3, SQ, D_MODEL), jnp.bfloat16),
            pltpu.VMEM((N_DEV, SQ, D_MODEL), jnp.float32),
            pltpu.VMEM((3, SQ, D_MODEL), jnp.bfloat16),
            pltpu.VMEM((3, SQ, D_MODEL), jnp.bfloat16),
            pltpu.SemaphoreType.DMA((3,)),
            pltpu.SemaphoreType.DMA((3,)),
            pltpu.SemaphoreType.DMA((3,)),
            pltpu.SemaphoreType.DMA((3,)),
        ],
        compiler_params=pltpu.CompilerParams(collective_id=0),
    )(x_bf, wq_bf, k_h, v_h, wo_bf)
    return out[None]


# device time: 94457 ns/iter; 1.4402x vs baseline; 1.4402x over previous
import functools

import jax
import jax.numpy as jnp
from jax import lax
from jax.experimental import pallas as pl
from jax.experimental.pallas import tpu as pltpu

N_DEV = 4
SQ = 256
D_MODEL = 1024
HG = 8
DH = 128
SKV = 4096
SCALE = 0.08838834764831843
NEG = -1e9
W = 512
G = 128


def kernel(x, Wq, K_ext, V_ext, Wo):
    my = lax.axis_index("i")

    x_bf = x[0].astype(jnp.bfloat16)
    wq_bf = Wq.astype(jnp.bfloat16)
    wo_bf = Wo.astype(jnp.bfloat16)
    k_sl = lax.dynamic_slice(K_ext, (0, 0, my * HG, 0), (1, SKV, HG, DH))[0]
    v_sl = lax.dynamic_slice(V_ext, (0, 0, my * HG, 0), (1, SKV, HG, DH))[0]
    k_h = jnp.transpose(k_sl, (1, 0, 2)).astype(jnp.bfloat16)
    v_h = jnp.transpose(v_sl, (1, 0, 2)).astype(jnp.bfloat16)

    def body(x_ref, wq_ref, k_ref, v_ref, wo_ref, out_ref,
             xbuf, partial, rs_sbuf, rs_rbuf,
             ag_ssem, ag_rsem, rs_ssem, rs_rsem):
        my_i = lax.axis_index("i")
        left = (my_i + N_DEV - 1) % N_DEV
        right = (my_i + 1) % N_DEV

        bsem = pltpu.get_barrier_semaphore()
        for nbr in (left, right):
            pl.semaphore_signal(bsem, inc=1, device_id=(nbr,),
                                device_id_type=pl.DeviceIdType.MESH)
        pl.semaphore_wait(bsem, 2)

        def ag_copy(h):
            src = x_ref if h == 0 else xbuf.at[h - 1]
            return pltpu.make_async_remote_copy(
                src_ref=src, dst_ref=xbuf.at[h],
                send_sem=ag_ssem.at[h], recv_sem=ag_rsem.at[h],
                device_id=(right,), device_id_type=pl.DeviceIdType.MESH)

        def rs_copy(t):
            return pltpu.make_async_remote_copy(
                src_ref=rs_sbuf.at[t], dst_ref=rs_rbuf.at[t],
                send_sem=rs_ssem.at[t], recv_sem=rs_rsem.at[t],
                device_id=(right,), device_id_type=pl.DeviceIdType.MESH)

        def softmax_rows(s, mask):
            s = jnp.where(mask, s, jnp.float32(NEG)) if mask is not None else s
            m = jnp.max(s, axis=1, keepdims=True)
            w = jnp.exp(s - m)
            d = jnp.sum(w, axis=1, keepdims=True)
            return (w / d).astype(jnp.bfloat16)

        def compute_partial(slot):
            q = (my_i + (N_DEV - slot)) % N_DEV
            xq = x_ref[:] if slot == 0 else xbuf[slot - 1]
            qm = jnp.dot(xq, wq_ref[:], preferred_element_type=jnp.float32)
            qm = (qm * SCALE).astype(jnp.bfloat16)
            start = pl.multiple_of(jnp.maximum(q * SQ - 128, 0), 128)
            qi = q * SQ + lax.broadcasted_iota(jnp.int32, (SQ, W), 0)
            kiw = start + lax.broadcasted_iota(jnp.int32, (SQ, W), 1)
            mask_w = (jnp.abs(qi - kiw) <= 128) | (kiw < 32)
            kig = lax.broadcasted_iota(jnp.int32, (SQ, G), 1)
            mask_g = (kig < 32) & (q > 0)
            mask = jnp.concatenate([mask_w, mask_g], axis=1)
            parts = []
            for h in range(HG):
                qh = qm[:, h * DH:(h + 1) * DH]
                kcat = jnp.concatenate(
                    [k_ref[h, pl.ds(start, W), :], k_ref[h, :G, :]], axis=0)
                vcat = jnp.concatenate(
                    [v_ref[h, pl.ds(start, W), :], v_ref[h, :G, :]], axis=0)
                s = lax.dot_general(qh, kcat, (((1,), (1,)), ((), ())),
                                    preferred_element_type=jnp.float32)
                w = softmax_rows(s, mask)
                parts.append(jnp.dot(w, vcat,
                                     preferred_element_type=jnp.float32))
            ctx = jnp.concatenate(parts, axis=1).astype(jnp.bfloat16)
            partial[slot] = jnp.dot(ctx, wo_ref[:],
                                    preferred_element_type=jnp.float32)

            @pl.when(q == 0)
            def _():
                parts32 = []
                for h in range(HG):
                    q32 = qm[0:32, h * DH:(h + 1) * DH]
                    s32 = lax.dot_general(q32, k_ref[h],
                                          (((1,), (1,)), ((), ())),
                                          preferred_element_type=jnp.float32)
                    w32 = softmax_rows(s32, None)
                    parts32.append(jnp.dot(w32, v_ref[h],
                                           preferred_element_type=jnp.float32))
                ctx32 = jnp.concatenate(parts32, axis=1).astype(jnp.bfloat16)
                partial[slot, 0:32, :] = jnp.dot(
                    ctx32, wo_ref[:], preferred_element_type=jnp.float32)

        ag0 = ag_copy(0)
        ag0.start()
        compute_partial(0)
        ag0.wait()

        ag1 = ag_copy(1)
        ag1.start()
        compute_partial(1)
        rs_sbuf[0] = partial[1].astype(jnp.bfloat16)
        rs0 = rs_copy(0)
        rs0.start()
        ag1.wait()

        ag2 = ag_copy(2)
        ag2.start()
        compute_partial(2)
        rs0.wait()
        partial[2] += rs_rbuf[0].astype(jnp.float32)
        rs_sbuf[1] = partial[2].astype(jnp.bfloat16)
        rs1 = rs_copy(1)
        rs1.start()
        ag2.wait()

        compute_partial(3)
        rs1.wait()
        partial[3] += rs_rbuf[1].astype(jnp.float32)
        rs_sbuf[2] = partial[3].astype(jnp.bfloat16)
        rs2 = rs_copy(2)
        rs2.start()
        rs2.wait()
        out_ref[:] = partial[0] + rs_rbuf[2].astype(jnp.float32)

        @functools.partial(pl.run_scoped, sem2=pltpu.SemaphoreType.REGULAR)
        def _(sem2):
            for nbr in (left, right):
                pl.semaphore_signal(sem2, inc=1, device_id=(nbr,),
                                    device_id_type=pl.DeviceIdType.MESH)
            pl.semaphore_wait(sem2, 2)

    out = pl.pallas_call(
        body,
        out_shape=jax.ShapeDtypeStruct((SQ, D_MODEL), jnp.float32),
        in_specs=[pl.BlockSpec(memory_space=pltpu.VMEM)] * 5,
        out_specs=pl.BlockSpec(memory_space=pltpu.VMEM),
        scratch_shapes=[
            pltpu.VMEM((3, SQ, D_MODEL), jnp.bfloat16),
            pltpu.VMEM((N_DEV, SQ, D_MODEL), jnp.float32),
            pltpu.VMEM((3, SQ, D_MODEL), jnp.bfloat16),
            pltpu.VMEM((3, SQ, D_MODEL), jnp.bfloat16),
            pltpu.SemaphoreType.DMA((3,)),
            pltpu.SemaphoreType.DMA((3,)),
            pltpu.SemaphoreType.DMA((3,)),
            pltpu.SemaphoreType.DMA((3,)),
        ],
        compiler_params=pltpu.CompilerParams(collective_id=0),
    )(x_bf, wq_bf, k_h, v_h, wo_bf)
    return out[None]


# device time: 73770 ns/iter; 1.8441x vs baseline; 1.2804x over previous
import functools

import jax
import jax.numpy as jnp
from jax import lax
from jax.experimental import pallas as pl
from jax.experimental.pallas import tpu as pltpu

N_DEV = 4
SQ = 256
D_MODEL = 1024
HG = 8
DH = 128
SKV = 4096
SCALE = 0.08838834764831843
NEG = -1e9
W = 512
G = 128


def kernel(x, Wq, K_ext, V_ext, Wo):
    my = lax.axis_index("i")

    x_bf = x[0].astype(jnp.bfloat16)
    wq_bf = Wq.astype(jnp.bfloat16)
    wo_bf = Wo.astype(jnp.bfloat16)

    def body(x_ref, wq_ref, k_hbm, v_hbm, wo_ref, out_ref,
             k_ref, v_ref, kvstage,
             xbuf, partial, rs_sbuf, rs_rbuf,
             dsem, ag_ssem, ag_rsem, rs_ssem, rs_rsem):
        my_i = lax.axis_index("i")
        left = (my_i + N_DEV - 1) % N_DEV
        right = (my_i + 1) % N_DEV

        def stage_head(h):
            buf = h % 2
            ck = pltpu.make_async_copy(
                k_hbm.at[0, :, my_i * HG + h, :], kvstage.at[buf, 0],
                dsem.at[buf, 0])
            cv = pltpu.make_async_copy(
                v_hbm.at[0, :, my_i * HG + h, :], kvstage.at[buf, 1],
                dsem.at[buf, 1])
            ck.start()
            cv.start()
            return ck, cv

        pend = {0: stage_head(0), 1: stage_head(1)}

        bsem = pltpu.get_barrier_semaphore()
        for nbr in (left, right):
            pl.semaphore_signal(bsem, inc=1, device_id=(nbr,),
                                device_id_type=pl.DeviceIdType.MESH)
        pl.semaphore_wait(bsem, 2)

        def ag_copy(h):
            src = x_ref if h == 0 else xbuf.at[h - 1]
            return pltpu.make_async_remote_copy(
                src_ref=src, dst_ref=xbuf.at[h],
                send_sem=ag_ssem.at[h], recv_sem=ag_rsem.at[h],
                device_id=(right,), device_id_type=pl.DeviceIdType.MESH)

        def rs_copy(t):
            return pltpu.make_async_remote_copy(
                src_ref=rs_sbuf.at[t], dst_ref=rs_rbuf.at[t],
                send_sem=rs_ssem.at[t], recv_sem=rs_rsem.at[t],
                device_id=(right,), device_id_type=pl.DeviceIdType.MESH)

        def softmax_rows(s, mask):
            s = jnp.where(mask, s, jnp.float32(NEG)) if mask is not None else s
            m = jnp.max(s, axis=1, keepdims=True)
            w = jnp.exp(s - m)
            d = jnp.sum(w, axis=1, keepdims=True)
            return (w / d).astype(jnp.bfloat16)

        def compute_partial(slot):
            q = (my_i + (N_DEV - slot)) % N_DEV
            xq = x_ref[:] if slot == 0 else xbuf[slot - 1]
            qm = jnp.dot(xq, wq_ref[:], preferred_element_type=jnp.float32)
            qm = (qm * SCALE).astype(jnp.bfloat16)
            start = pl.multiple_of(jnp.maximum(q * SQ - 128, 0), 128)
            qi = q * SQ + lax.broadcasted_iota(jnp.int32, (SQ, W), 0)
            kiw = start + lax.broadcasted_iota(jnp.int32, (SQ, W), 1)
            mask_w = (jnp.abs(qi - kiw) <= 128) | (kiw < 32)
            kig = lax.broadcasted_iota(jnp.int32, (SQ, G), 1)
            mask_g = (kig < 32) & (q > 0)
            mask = jnp.concatenate([mask_w, mask_g], axis=1)
            parts = []
            for h in range(HG):
                qh = qm[:, h * DH:(h + 1) * DH]
                kcat = jnp.concatenate(
                    [k_ref[h, pl.ds(start, W), :], k_ref[h, :G, :]], axis=0)
                vcat = jnp.concatenate(
                    [v_ref[h, pl.ds(start, W), :], v_ref[h, :G, :]], axis=0)
                s = lax.dot_general(qh, kcat, (((1,), (1,)), ((), ())),
                                    preferred_element_type=jnp.float32)
                w = softmax_rows(s, mask)
                parts.append(jnp.dot(w, vcat,
                                     preferred_element_type=jnp.float32))
            ctx = jnp.concatenate(parts, axis=1).astype(jnp.bfloat16)
            partial[slot] = jnp.dot(ctx, wo_ref[:],
                                    preferred_element_type=jnp.float32)

            @pl.when(q == 0)
            def _():
                parts32 = []
                for h in range(HG):
                    q32 = qm[0:32, h * DH:(h + 1) * DH]
                    s32 = lax.dot_general(q32, k_ref[h],
                                          (((1,), (1,)), ((), ())),
                                          preferred_element_type=jnp.float32)
                    w32 = softmax_rows(s32, None)
                    parts32.append(jnp.dot(w32, v_ref[h],
                                           preferred_element_type=jnp.float32))
                ctx32 = jnp.concatenate(parts32, axis=1).astype(jnp.bfloat16)
                partial[slot, 0:32, :] = jnp.dot(
                    ctx32, wo_ref[:], preferred_element_type=jnp.float32)

        ag0 = ag_copy(0)
        ag0.start()

        for h in range(HG):
            ck, cv = pend[h]
            ck.wait()
            cv.wait()
            k_ref[h] = kvstage[h % 2, 0].astype(jnp.bfloat16)
            v_ref[h] = kvstage[h % 2, 1].astype(jnp.bfloat16)
            if h + 2 < HG:
                pend[h + 2] = stage_head(h + 2)

        compute_partial(0)
        ag0.wait()

        ag1 = ag_copy(1)
        ag1.start()
        compute_partial(1)
        rs_sbuf[0] = partial[1].astype(jnp.bfloat16)
        rs0 = rs_copy(0)
        rs0.start()
        ag1.wait()

        ag2 = ag_copy(2)
        ag2.start()
        compute_partial(2)
        rs0.wait()
        partial[2] += rs_rbuf[0].astype(jnp.float32)
        rs_sbuf[1] = partial[2].astype(jnp.bfloat16)
        rs1 = rs_copy(1)
        rs1.start()
        ag2.wait()

        compute_partial(3)
        rs1.wait()
        partial[3] += rs_rbuf[1].astype(jnp.float32)
        rs_sbuf[2] = partial[3].astype(jnp.bfloat16)
        rs2 = rs_copy(2)
        rs2.start()
        rs2.wait()
        out_ref[:] = partial[0] + rs_rbuf[2].astype(jnp.float32)

        @functools.partial(pl.run_scoped, sem2=pltpu.SemaphoreType.REGULAR)
        def _(sem2):
            for nbr in (left, right):
                pl.semaphore_signal(sem2, inc=1, device_id=(nbr,),
                                    device_id_type=pl.DeviceIdType.MESH)
            pl.semaphore_wait(sem2, 2)

    out = pl.pallas_call(
        body,
        out_shape=jax.ShapeDtypeStruct((SQ, D_MODEL), jnp.float32),
        in_specs=[
            pl.BlockSpec(memory_space=pltpu.VMEM),
            pl.BlockSpec(memory_space=pltpu.VMEM),
            pl.BlockSpec(memory_space=pl.ANY),
            pl.BlockSpec(memory_space=pl.ANY),
            pl.BlockSpec(memory_space=pltpu.VMEM),
        ],
        out_specs=pl.BlockSpec(memory_space=pltpu.VMEM),
        scratch_shapes=[
            pltpu.VMEM((HG, SKV, DH), jnp.bfloat16),
            pltpu.VMEM((HG, SKV, DH), jnp.bfloat16),
            pltpu.VMEM((2, 2, SKV, DH), jnp.float32),
            pltpu.VMEM((3, SQ, D_MODEL), jnp.bfloat16),
            pltpu.VMEM((N_DEV, SQ, D_MODEL), jnp.float32),
            pltpu.VMEM((3, SQ, D_MODEL), jnp.bfloat16),
            pltpu.VMEM((3, SQ, D_MODEL), jnp.bfloat16),
            pltpu.SemaphoreType.DMA((2, 2)),
            pltpu.SemaphoreType.DMA((3,)),
            pltpu.SemaphoreType.DMA((3,)),
            pltpu.SemaphoreType.DMA((3,)),
            pltpu.SemaphoreType.DMA((3,)),
        ],
        compiler_params=pltpu.CompilerParams(
            collective_id=0, vmem_limit_bytes=60 * 1024 * 1024),
    )(x_bf, wq_bf, K_ext, V_ext, wo_bf)
    return out[None]
